# baseline (speedup 1.0000x reference)
import jax
import jax.numpy as jnp
from jax.experimental import pallas as pl
from jax.experimental.pallas import tpu as pltpu

N = 100000
NODE_DIM = 128
HIDDEN_DIM = 64
NUM_GRAPHS = 64
BLOCK = 5000
NBLK = N // BLOCK


def _pool_kernel(h_ref, b3_ref, w1_ref, b1_ref, w2_ref,
                 out_ref, acc_num, acc_den):
    i = pl.program_id(0)

    @pl.when(i == 0)
    def _init():
        acc_num[...] = jnp.zeros_like(acc_num)
        acc_den[...] = jnp.zeros_like(acc_den)

    h = h_ref[...]
    hb = h.astype(jnp.bfloat16)
    hid = jax.lax.dot_general(hb, w1_ref[...],
                              (((1,), (1,)), ((), ())),
                              preferred_element_type=jnp.float32)
    hid = jnp.tanh(hid + b1_ref[...])
    s = jax.lax.dot_general(hid.astype(jnp.bfloat16), w2_ref[...],
                            (((1,), (0,)), ((), ())),
                            preferred_element_type=jnp.float32)
    sT = s[:, :8].T
    eT = jnp.exp(sT)

    gids = jax.lax.broadcasted_iota(jnp.int32, (NUM_GRAPHS, BLOCK), 0)
    b_row = b3_ref[0, :, :]
    e_row = jax.lax.broadcast_in_dim(eT[:1, :], (NUM_GRAPHS, BLOCK),
                                     (0, 1))
    ohe = jnp.where(gids == b_row, e_row, 0.0).astype(jnp.bfloat16)

    acc_num[...] += jax.lax.dot_general(ohe, hb,
                                        (((1,), (0,)), ((), ())),
                                        preferred_element_type=jnp.float32)
    ones8 = jnp.ones((8, BLOCK), dtype=jnp.bfloat16)
    den8 = jax.lax.dot_general(ones8, ohe,
                               (((1,), (1,)), ((), ())),
                               preferred_element_type=jnp.float32)
    acc_den[...] += den8.T

    @pl.when(i == NBLK - 1)
    def _finish():
        ones = jnp.full((8, NODE_DIM), 0.125, dtype=jnp.float32)
        den = jax.lax.dot_general(acc_den[...], ones,
                                  (((1,), (0,)), ((), ())),
                                  preferred_element_type=jnp.float32)
        den = jnp.where(den == 0.0, 1.0, den)
        out_ref[...] = acc_num[...] / den


@jax.jit
def _pooled(h, batch_i32, W1, b1, W2):
    b3 = batch_i32.reshape(NBLK, 1, BLOCK)
    b1r = b1.reshape(1, HIDDEN_DIM)
    w1b = W1.astype(jnp.bfloat16)
    w2rep = jnp.broadcast_to(W2.reshape(HIDDEN_DIM, 1),
                             (HIDDEN_DIM, NODE_DIM)).astype(jnp.bfloat16)
    in_specs = [
        pl.BlockSpec((BLOCK, NODE_DIM), lambda i: (i, 0)),
        pl.BlockSpec((1, 1, BLOCK), lambda i: (i, 0, 0)),
        pl.BlockSpec((HIDDEN_DIM, NODE_DIM), lambda i: (0, 0)),
        pl.BlockSpec((1, HIDDEN_DIM), lambda i: (0, 0)),
        pl.BlockSpec((HIDDEN_DIM, NODE_DIM), lambda i: (0, 0)),
    ]
    return pl.pallas_call(
        _pool_kernel,
        grid=(NBLK,),
        in_specs=in_specs,
        out_specs=pl.BlockSpec((NUM_GRAPHS, NODE_DIM), lambda i: (0, 0)),
        out_shape=jax.ShapeDtypeStruct((NUM_GRAPHS, NODE_DIM), jnp.float32),
        scratch_shapes=[
            pltpu.VMEM((NUM_GRAPHS, NODE_DIM), jnp.float32),
            pltpu.VMEM((NUM_GRAPHS, 8), jnp.float32),
        ],
        compiler_params=pltpu.CompilerParams(
            dimension_semantics=("arbitrary",),
        ),
    )(h, b3, W1, b1r, w2rep)


def kernel(h, batch, W1, b1, W2, b2):
    del b2
    return _pooled(h, batch.astype(jnp.int32), W1, b1, W2)


# speedup vs baseline: 1.4622x; 1.4622x over previous
import jax
import jax.numpy as jnp
from jax.experimental import pallas as pl
from jax.experimental.pallas import tpu as pltpu

N = 100000
NODE_DIM = 128
HIDDEN_DIM = 64
NUM_GRAPHS = 64
BLOCK = 5000
NBLK = N // BLOCK


def _pool_kernel(h_ref, b3_ref, w1_ref, b1_ref, w2_ref,
                 out_ref, acc_num, acc_den):
    i = pl.program_id(0)

    @pl.when(i == 0)
    def _init():
        acc_num[...] = jnp.zeros_like(acc_num)
        acc_den[...] = jnp.zeros_like(acc_den)

    h = h_ref[...]
    hb = h.astype(jnp.bfloat16)
    hidT = jax.lax.dot_general(w1_ref[...], hb,
                               (((1,), (1,)), ((), ())),
                               preferred_element_type=jnp.float32)
    hidT = jnp.tanh(hidT + b1_ref[...])
    sT = jax.lax.dot_general(w2_ref[...], hidT.astype(jnp.bfloat16),
                             (((1,), (0,)), ((), ())),
                             preferred_element_type=jnp.float32)
    eT = jnp.exp(sT)

    gids = jax.lax.broadcasted_iota(jnp.int32, (NUM_GRAPHS, BLOCK), 0)
    b_row = b3_ref[0, :, :]
    e_row = jax.lax.broadcast_in_dim(eT[:1, :], (NUM_GRAPHS, BLOCK),
                                     (0, 1))
    ohe = jnp.where(gids == b_row, e_row, 0.0).astype(jnp.bfloat16)

    acc_num[...] += jax.lax.dot_general(ohe, hb,
                                        (((1,), (0,)), ((), ())),
                                        preferred_element_type=jnp.float32)
    ones8 = jnp.ones((8, BLOCK), dtype=jnp.bfloat16)
    acc_den[...] += jax.lax.dot_general(ones8, ohe,
                                        (((1,), (1,)), ((), ())),
                                        preferred_element_type=jnp.float32)

    @pl.when(i == NBLK - 1)
    def _finish():
        denT = acc_den[...]
        den8 = denT.T
        ones = jnp.full((8, NODE_DIM), 0.125, dtype=jnp.float32)
        den = jax.lax.dot_general(den8, ones,
                                  (((1,), (0,)), ((), ())),
                                  preferred_element_type=jnp.float32)
        den = jnp.where(den == 0.0, 1.0, den)
        out_ref[...] = acc_num[...] / den


@jax.jit
def _pooled(h, batch_i32, W1, b1, W2):
    b3 = batch_i32.reshape(NBLK, 1, BLOCK)
    b1bc = jnp.broadcast_to(b1.reshape(HIDDEN_DIM, 1),
                            (HIDDEN_DIM, BLOCK))
    w1b = W1.astype(jnp.bfloat16)
    w2rep8 = jnp.broadcast_to(W2.reshape(1, HIDDEN_DIM),
                              (8, HIDDEN_DIM)).astype(jnp.bfloat16)
    in_specs = [
        pl.BlockSpec((BLOCK, NODE_DIM), lambda i: (i, 0)),
        pl.BlockSpec((1, 1, BLOCK), lambda i: (i, 0, 0)),
        pl.BlockSpec((HIDDEN_DIM, NODE_DIM), lambda i: (0, 0)),
        pl.BlockSpec((HIDDEN_DIM, BLOCK), lambda i: (0, 0)),
        pl.BlockSpec((8, HIDDEN_DIM), lambda i: (0, 0)),
    ]
    return pl.pallas_call(
        _pool_kernel,
        grid=(NBLK,),
        in_specs=in_specs,
        out_specs=pl.BlockSpec((NUM_GRAPHS, NODE_DIM), lambda i: (0, 0)),
        out_shape=jax.ShapeDtypeStruct((NUM_GRAPHS, NODE_DIM), jnp.float32),
        scratch_shapes=[
            pltpu.VMEM((NUM_GRAPHS, NODE_DIM), jnp.float32),
            pltpu.VMEM((8, NUM_GRAPHS), jnp.float32),
        ],
        compiler_params=pltpu.CompilerParams(
            dimension_semantics=("arbitrary",),
        ),
    )(h, b3, w1b, b1bc, w2rep8)


def kernel(h, batch, W1, b1, W2, b2):
    del b2
    return _pooled(h, batch.astype(jnp.int32), W1, b1, W2)


# speedup vs baseline: 1.6076x; 1.0995x over previous
import jax
import jax.numpy as jnp
from jax.experimental import pallas as pl
from jax.experimental.pallas import tpu as pltpu

N = 100000
NODE_DIM = 128
HIDDEN_DIM = 64
NUM_GRAPHS = 64
BLOCK = 10000
NBLK = N // BLOCK
CHUNK = 5000
NCHUNK = BLOCK // CHUNK


def _pool_kernel(h_ref, b3_ref, w1_ref, b1_ref, w2_ref, gids_ref,
                 out_ref, acc_num, acc_den):
    i = pl.program_id(0)

    @pl.when(i == 0)
    def _init():
        acc_num[...] = jnp.zeros_like(acc_num)
        acc_den[...] = jnp.zeros_like(acc_den)

    gids = gids_ref[...]
    b1b = b1_ref[...]
    ones8 = jnp.ones((8, CHUNK), dtype=jnp.bfloat16)
    num_parts = []
    den_parts = []
    for c in range(NCHUNK):
        h = h_ref[pl.ds(c * CHUNK, CHUNK), :]
        hb = h.astype(jnp.bfloat16)
        hidT = jax.lax.dot_general(w1_ref[...], hb,
                                   (((1,), (1,)), ((), ())),
                                   preferred_element_type=jnp.float32)
        hidT = jnp.tanh(hidT.astype(jnp.bfloat16) + b1b)
        sT = jax.lax.dot_general(w2_ref[...], hidT,
                                 (((1,), (0,)), ((), ())),
                                 preferred_element_type=jnp.float32)
        eT = jnp.exp(sT)

        b_row = b3_ref[0, :, pl.ds(c * CHUNK, CHUNK)]
        e_row = jax.lax.broadcast_in_dim(eT[:1, :].astype(jnp.bfloat16),
                                         (NUM_GRAPHS, CHUNK), (0, 1))
        ohe = jnp.where(gids == b_row, e_row, jnp.bfloat16(0.0))

        num_parts.append(jax.lax.dot_general(
            ohe, hb, (((1,), (0,)), ((), ())),
            preferred_element_type=jnp.float32))
        den_parts.append(jax.lax.dot_general(
            ones8, ohe, (((1,), (1,)), ((), ())),
            preferred_element_type=jnp.float32))

    acc_num[...] += sum(num_parts)
    acc_den[...] += sum(den_parts)

    @pl.when(i == NBLK - 1)
    def _finish():
        denT = acc_den[...]
        den8 = denT.T
        ones = jnp.full((8, NODE_DIM), 0.125, dtype=jnp.float32)
        den = jax.lax.dot_general(den8, ones,
                                  (((1,), (0,)), ((), ())),
                                  preferred_element_type=jnp.float32)
        den = jnp.where(den == 0.0, 1.0, den)
        out_ref[...] = acc_num[...] / den


@jax.jit
def _pooled(h, batch_i32, W1, b1, W2):
    b3 = batch_i32.astype(jnp.bfloat16).reshape(NBLK, 1, BLOCK)
    b1bc = jnp.broadcast_to(b1.reshape(HIDDEN_DIM, 1),
                            (HIDDEN_DIM, CHUNK)).astype(jnp.bfloat16)
    w1b = W1.astype(jnp.bfloat16)
    w2rep8 = jnp.broadcast_to(W2.reshape(1, HIDDEN_DIM),
                              (8, HIDDEN_DIM)).astype(jnp.bfloat16)
    gidsb = jnp.broadcast_to(
        jnp.arange(NUM_GRAPHS, dtype=jnp.float32).reshape(NUM_GRAPHS, 1),
        (NUM_GRAPHS, CHUNK)).astype(jnp.bfloat16)
    in_specs = [
        pl.BlockSpec((BLOCK, NODE_DIM), lambda i: (i, 0)),
        pl.BlockSpec((1, 1, BLOCK), lambda i: (i, 0, 0)),
        pl.BlockSpec((HIDDEN_DIM, NODE_DIM), lambda i: (0, 0)),
        pl.BlockSpec((HIDDEN_DIM, CHUNK), lambda i: (0, 0)),
        pl.BlockSpec((8, HIDDEN_DIM), lambda i: (0, 0)),
        pl.BlockSpec((NUM_GRAPHS, CHUNK), lambda i: (0, 0)),
    ]
    return pl.pallas_call(
        _pool_kernel,
        grid=(NBLK,),
        in_specs=in_specs,
        out_specs=pl.BlockSpec((NUM_GRAPHS, NODE_DIM), lambda i: (0, 0)),
        out_shape=jax.ShapeDtypeStruct((NUM_GRAPHS, NODE_DIM), jnp.float32),
        scratch_shapes=[
            pltpu.VMEM((NUM_GRAPHS, NODE_DIM), jnp.float32),
            pltpu.VMEM((8, NUM_GRAPHS), jnp.float32),
        ],
        compiler_params=pltpu.CompilerParams(
            dimension_semantics=("arbitrary",),
        ),
    )(h, b3, w1b, b1bc, w2rep8, gidsb)


def kernel(h, batch, W1, b1, W2, b2):
    del b2
    return _pooled(h, batch.astype(jnp.int32), W1, b1, W2)


# speedup vs baseline: 1.7392x; 1.0819x over previous
import jax
import jax.numpy as jnp
from jax.experimental import pallas as pl
from jax.experimental.pallas import tpu as pltpu

N = 100000
NODE_DIM = 128
HIDDEN_DIM = 64
NUM_GRAPHS = 64
BLOCK = 10000
NBLK = N // BLOCK
CHUNK = 5000
NCHUNK = BLOCK // CHUNK


def _pool_kernel(h_ref, b3_ref, w1_ref, b1_ref, w2_ref, gids_ref,
                 out_ref, acc_num, acc_den):
    i = pl.program_id(0)

    @pl.when(i == 0)
    def _init():
        acc_num[...] = jnp.zeros_like(acc_num)
        acc_den[...] = jnp.zeros_like(acc_den)

    gids = gids_ref[...]
    b1b = b1_ref[...]
    w1b = w1_ref[...].astype(jnp.bfloat16)
    w2b = jnp.broadcast_to(w2_ref[...], (8, HIDDEN_DIM)
                           ).astype(jnp.bfloat16)
    ones8 = jnp.ones((8, CHUNK), dtype=jnp.bfloat16)
    num_parts = []
    den_parts = []
    for c in range(NCHUNK):
        h = h_ref[pl.ds(c * CHUNK, CHUNK), :]
        hb = h.astype(jnp.bfloat16)
        hidT = jax.lax.dot_general(w1b, hb,
                                   (((1,), (1,)), ((), ())),
                                   preferred_element_type=jnp.float32)
        hidT = jnp.tanh(hidT.astype(jnp.bfloat16) + b1b)
        sT = jax.lax.dot_general(w2b, hidT,
                                 (((1,), (0,)), ((), ())),
                                 preferred_element_type=jnp.float32)
        eT = jnp.exp(sT)

        b_row = b3_ref[0, :, pl.ds(c * CHUNK, CHUNK)].astype(
            jnp.bfloat16)
        e_row = jax.lax.broadcast_in_dim(eT[:1, :].astype(jnp.bfloat16),
                                         (NUM_GRAPHS, CHUNK), (0, 1))
        ohe = jnp.where(gids == b_row, e_row, jnp.bfloat16(0.0))

        num_parts.append(jax.lax.dot_general(
            ohe, hb, (((1,), (0,)), ((), ())),
            preferred_element_type=jnp.float32))
        den_parts.append(jax.lax.dot_general(
            ones8, ohe, (((1,), (1,)), ((), ())),
            preferred_element_type=jnp.float32))

    acc_num[...] += sum(num_parts)
    acc_den[...] += sum(den_parts)

    @pl.when(i == NBLK - 1)
    def _finish():
        denT = acc_den[...]
        den8 = denT.T
        ones = jnp.full((8, NODE_DIM), 0.125, dtype=jnp.float32)
        den = jax.lax.dot_general(den8, ones,
                                  (((1,), (0,)), ((), ())),
                                  preferred_element_type=jnp.float32)
        den = jnp.where(den == 0.0, 1.0, den)
        out_ref[...] = acc_num[...] / den


@jax.jit
def _pooled(h, batch_i32, W1, b1, W2):
    b3 = batch_i32.reshape(NBLK, 1, BLOCK)
    b1bc = jnp.broadcast_to(b1.reshape(HIDDEN_DIM, 1),
                            (HIDDEN_DIM, CHUNK)).astype(jnp.bfloat16)
    w2r = W2.reshape(1, HIDDEN_DIM)
    gidsb = jnp.broadcast_to(
        jnp.arange(NUM_GRAPHS, dtype=jnp.float32).reshape(NUM_GRAPHS, 1),
        (NUM_GRAPHS, CHUNK)).astype(jnp.bfloat16)
    in_specs = [
        pl.BlockSpec((BLOCK, NODE_DIM), lambda i: (i, 0)),
        pl.BlockSpec((1, 1, BLOCK), lambda i: (i, 0, 0)),
        pl.BlockSpec((HIDDEN_DIM, NODE_DIM), lambda i: (0, 0)),
        pl.BlockSpec((HIDDEN_DIM, CHUNK), lambda i: (0, 0)),
        pl.BlockSpec((1, HIDDEN_DIM), lambda i: (0, 0)),
        pl.BlockSpec((NUM_GRAPHS, CHUNK), lambda i: (0, 0)),
    ]
    return pl.pallas_call(
        _pool_kernel,
        grid=(NBLK,),
        in_specs=in_specs,
        out_specs=pl.BlockSpec((NUM_GRAPHS, NODE_DIM), lambda i: (0, 0)),
        out_shape=jax.ShapeDtypeStruct((NUM_GRAPHS, NODE_DIM), jnp.float32),
        scratch_shapes=[
            pltpu.VMEM((NUM_GRAPHS, NODE_DIM), jnp.float32),
            pltpu.VMEM((8, NUM_GRAPHS), jnp.float32),
        ],
        compiler_params=pltpu.CompilerParams(
            dimension_semantics=("arbitrary",),
        ),
    )(h, b3, W1, b1bc, w2r, gidsb)


def kernel(h, batch, W1, b1, W2, b2):
    del b2
    return _pooled(h, batch.astype(jnp.int32), W1, b1, W2)


# speedup vs baseline: 1.7776x; 1.0221x over previous
import jax
import jax.numpy as jnp
from jax.experimental import pallas as pl
from jax.experimental.pallas import tpu as pltpu

N = 100000
NODE_DIM = 128
HIDDEN_DIM = 64
NUM_GRAPHS = 64
BLOCK = 25000
NBLK = N // BLOCK
CHUNK = 5000
NCHUNK = BLOCK // CHUNK


def _pool_kernel(h_ref, b3_ref, w1_ref, b1_ref, w2_ref, gids_ref,
                 out_ref, acc_num, acc_den):
    i = pl.program_id(0)

    @pl.when(i == 0)
    def _init():
        acc_num[...] = jnp.zeros_like(acc_num)
        acc_den[...] = jnp.zeros_like(acc_den)

    gids = gids_ref[...]
    b1b = b1_ref[...]
    w1b = w1_ref[...].astype(jnp.bfloat16)
    w2b = jnp.broadcast_to(w2_ref[...], (8, HIDDEN_DIM)
                           ).astype(jnp.bfloat16)
    ones8 = jnp.ones((8, CHUNK), dtype=jnp.bfloat16)
    num_parts = []
    den_parts = []
    for c in range(NCHUNK):
        h = h_ref[pl.ds(c * CHUNK, CHUNK), :]
        hb = h.astype(jnp.bfloat16)
        hidT = jax.lax.dot_general(w1b, hb,
                                   (((1,), (1,)), ((), ())),
                                   preferred_element_type=jnp.float32)
        hidT = jnp.tanh(hidT.astype(jnp.bfloat16) + b1b)
        sT = jax.lax.dot_general(w2b, hidT,
                                 (((1,), (0,)), ((), ())),
                                 preferred_element_type=jnp.float32)
        eT = jnp.exp(sT)

        b_row = b3_ref[0, :, pl.ds(c * CHUNK, CHUNK)].astype(
            jnp.bfloat16)
        e_row = jax.lax.broadcast_in_dim(eT[:1, :].astype(jnp.bfloat16),
                                         (NUM_GRAPHS, CHUNK), (0, 1))
        ohe = jnp.where(gids == b_row, e_row, jnp.bfloat16(0.0))

        num_parts.append(jax.lax.dot_general(
            ohe, hb, (((1,), (0,)), ((), ())),
            preferred_element_type=jnp.float32))
        den_parts.append(jax.lax.dot_general(
            ones8, ohe, (((1,), (1,)), ((), ())),
            preferred_element_type=jnp.float32))

    acc_num[...] += sum(num_parts)
    acc_den[...] += sum(den_parts)

    @pl.when(i == NBLK - 1)
    def _finish():
        denT = acc_den[...]
        den8 = denT.T
        ones = jnp.full((8, NODE_DIM), 0.125, dtype=jnp.float32)
        den = jax.lax.dot_general(den8, ones,
                                  (((1,), (0,)), ((), ())),
                                  preferred_element_type=jnp.float32)
        den = jnp.where(den == 0.0, 1.0, den)
        out_ref[...] = acc_num[...] / den


@jax.jit
def _pooled(h, batch_i32, W1, b1, W2):
    b3 = batch_i32.reshape(NBLK, 1, BLOCK)
    b1bc = jnp.broadcast_to(b1.reshape(HIDDEN_DIM, 1),
                            (HIDDEN_DIM, CHUNK)).astype(jnp.bfloat16)
    w2r = W2.reshape(1, HIDDEN_DIM)
    gidsb = jnp.broadcast_to(
        jnp.arange(NUM_GRAPHS, dtype=jnp.float32).reshape(NUM_GRAPHS, 1),
        (NUM_GRAPHS, CHUNK)).astype(jnp.bfloat16)
    in_specs = [
        pl.BlockSpec((BLOCK, NODE_DIM), lambda i: (i, 0)),
        pl.BlockSpec((1, 1, BLOCK), lambda i: (i, 0, 0)),
        pl.BlockSpec((HIDDEN_DIM, NODE_DIM), lambda i: (0, 0)),
        pl.BlockSpec((HIDDEN_DIM, CHUNK), lambda i: (0, 0)),
        pl.BlockSpec((1, HIDDEN_DIM), lambda i: (0, 0)),
        pl.BlockSpec((NUM_GRAPHS, CHUNK), lambda i: (0, 0)),
    ]
    return pl.pallas_call(
        _pool_kernel,
        grid=(NBLK,),
        in_specs=in_specs,
        out_specs=pl.BlockSpec((NUM_GRAPHS, NODE_DIM), lambda i: (0, 0)),
        out_shape=jax.ShapeDtypeStruct((NUM_GRAPHS, NODE_DIM), jnp.float32),
        scratch_shapes=[
            pltpu.VMEM((NUM_GRAPHS, NODE_DIM), jnp.float32),
            pltpu.VMEM((8, NUM_GRAPHS), jnp.float32),
        ],
        compiler_params=pltpu.CompilerParams(
            dimension_semantics=("arbitrary",),
        ),
    )(h, b3, W1, b1bc, w2r, gidsb)


def kernel(h, batch, W1, b1, W2, b2):
    del b2
    return _pooled(h, batch.astype(jnp.int32), W1, b1, W2)


# speedup vs baseline: 1.7947x; 1.0096x over previous
import jax
import jax.numpy as jnp
from jax.experimental import pallas as pl
from jax.experimental.pallas import tpu as pltpu

N = 100000
NODE_DIM = 128
HIDDEN_DIM = 64
NUM_GRAPHS = 64
BLOCK = 20000
NBLK = N // BLOCK
CHUNK = 5000
NCHUNK = BLOCK // CHUNK


def _pool_kernel(h_ref, b3_ref, w1_ref, b1_ref, w2_ref, gids_ref,
                 out_ref, acc_num, acc_den):
    i = pl.program_id(0)

    @pl.when(i == 0)
    def _init():
        acc_num[...] = jnp.zeros_like(acc_num)
        acc_den[...] = jnp.zeros_like(acc_den)

    gids = gids_ref[...]
    b1b = b1_ref[...]
    w1b = w1_ref[...].astype(jnp.bfloat16)
    w2b = jnp.broadcast_to(w2_ref[...], (8, HIDDEN_DIM)
                           ).astype(jnp.bfloat16)
    ones8 = jnp.ones((8, CHUNK), dtype=jnp.bfloat16)
    num_parts = []
    den_parts = []
    for c in range(NCHUNK):
        h = h_ref[pl.ds(c * CHUNK, CHUNK), :]
        hb = h.astype(jnp.bfloat16)
        hidT = jax.lax.dot_general(w1b, hb,
                                   (((1,), (1,)), ((), ())),
                                   preferred_element_type=jnp.float32)
        hidT = jnp.tanh(hidT.astype(jnp.bfloat16) + b1b)
        sT = jax.lax.dot_general(w2b, hidT,
                                 (((1,), (0,)), ((), ())),
                                 preferred_element_type=jnp.float32)
        eT = jnp.exp(sT)

        b_row = b3_ref[0, :, pl.ds(c * CHUNK, CHUNK)].astype(
            jnp.bfloat16)
        e_row = jax.lax.broadcast_in_dim(eT[:1, :].astype(jnp.bfloat16),
                                         (NUM_GRAPHS, CHUNK), (0, 1))
        ohe = jnp.where(gids == b_row, e_row, jnp.bfloat16(0.0))

        num_parts.append(jax.lax.dot_general(
            ohe, hb, (((1,), (0,)), ((), ())),
            preferred_element_type=jnp.float32))
        den_parts.append(jax.lax.dot_general(
            ones8, ohe, (((1,), (1,)), ((), ())),
            preferred_element_type=jnp.float32))

    acc_num[...] += sum(num_parts)
    acc_den[...] += sum(den_parts)

    @pl.when(i == NBLK - 1)
    def _finish():
        denT = acc_den[...]
        den8 = denT.T
        ones = jnp.full((8, NODE_DIM), 0.125, dtype=jnp.float32)
        den = jax.lax.dot_general(den8, ones,
                                  (((1,), (0,)), ((), ())),
                                  preferred_element_type=jnp.float32)
        den = jnp.where(den == 0.0, 1.0, den)
        out_ref[...] = acc_num[...] / den


@jax.jit
def _pooled(h, batch_i32, W1, b1, W2):
    b3 = batch_i32.reshape(NBLK, 1, BLOCK)
    b1bc = jnp.broadcast_to(b1.reshape(HIDDEN_DIM, 1),
                            (HIDDEN_DIM, CHUNK)).astype(jnp.bfloat16)
    w2r = W2.reshape(1, HIDDEN_DIM)
    gidsb = jnp.broadcast_to(
        jnp.arange(NUM_GRAPHS, dtype=jnp.float32).reshape(NUM_GRAPHS, 1),
        (NUM_GRAPHS, CHUNK)).astype(jnp.bfloat16)
    in_specs = [
        pl.BlockSpec((BLOCK, NODE_DIM), lambda i: (i, 0)),
        pl.BlockSpec((1, 1, BLOCK), lambda i: (i, 0, 0)),
        pl.BlockSpec((HIDDEN_DIM, NODE_DIM), lambda i: (0, 0)),
        pl.BlockSpec((HIDDEN_DIM, CHUNK), lambda i: (0, 0)),
        pl.BlockSpec((1, HIDDEN_DIM), lambda i: (0, 0)),
        pl.BlockSpec((NUM_GRAPHS, CHUNK), lambda i: (0, 0)),
    ]
    return pl.pallas_call(
        _pool_kernel,
        grid=(NBLK,),
        in_specs=in_specs,
        out_specs=pl.BlockSpec((NUM_GRAPHS, NODE_DIM), lambda i: (0, 0)),
        out_shape=jax.ShapeDtypeStruct((NUM_GRAPHS, NODE_DIM), jnp.float32),
        scratch_shapes=[
            pltpu.VMEM((NUM_GRAPHS, NODE_DIM), jnp.float32),
            pltpu.VMEM((8, NUM_GRAPHS), jnp.float32),
        ],
        compiler_params=pltpu.CompilerParams(
            dimension_semantics=("arbitrary",),
        ),
    )(h, b3, W1, b1bc, w2r, gidsb)


def kernel(h, batch, W1, b1, W2, b2):
    del b2
    return _pooled(h, batch.astype(jnp.int32), W1, b1, W2)


# speedup vs baseline: 1.8824x; 1.0489x over previous
import jax
import jax.numpy as jnp
from jax.experimental import pallas as pl
from jax.experimental.pallas import tpu as pltpu

N = 100000
NODE_DIM = 128
HIDDEN_DIM = 64
NUM_GRAPHS = 64
BLOCK = 20000
NBLK = N // BLOCK
CHUNK = 5000
NCHUNK = BLOCK // CHUNK


def _pool_kernel(h_ref, b3_ref, w1_ref, w2_ref, cst_ref,
                 out_ref, acc_num, acc_den):
    i = pl.program_id(0)

    @pl.when(i == 0)
    def _init():
        acc_num[...] = jnp.zeros_like(acc_num)
        acc_den[...] = jnp.zeros_like(acc_den)

    gids = cst_ref[:NUM_GRAPHS, :]
    b1b = cst_ref[NUM_GRAPHS:, :]
    w1b = w1_ref[...].astype(jnp.bfloat16)
    w2b = jnp.broadcast_to(w2_ref[...], (8, HIDDEN_DIM)
                           ).astype(jnp.bfloat16)
    ones8 = jnp.ones((8, CHUNK), dtype=jnp.bfloat16)
    num_parts = []
    den_parts = []
    for c in range(NCHUNK):
        h = h_ref[pl.ds(c * CHUNK, CHUNK), :]
        hb = h.astype(jnp.bfloat16)
        hidT = jax.lax.dot_general(w1b, hb,
                                   (((1,), (1,)), ((), ())),
                                   preferred_element_type=jnp.float32)
        hidT = jnp.tanh(hidT.astype(jnp.bfloat16) + b1b)
        sT = jax.lax.dot_general(w2b, hidT,
                                 (((1,), (0,)), ((), ())),
                                 preferred_element_type=jnp.float32)
        eT = jnp.exp(sT)

        b_row = b3_ref[0, :, pl.ds(c * CHUNK, CHUNK)].astype(
            jnp.bfloat16)
        e_row = jax.lax.broadcast_in_dim(eT[:1, :].astype(jnp.bfloat16),
                                         (NUM_GRAPHS, CHUNK), (0, 1))
        ohe = jnp.where(gids == b_row, e_row, jnp.bfloat16(0.0))

        num_parts.append(jax.lax.dot_general(
            ohe, hb, (((1,), (0,)), ((), ())),
            preferred_element_type=jnp.float32))
        den_parts.append(jax.lax.dot_general(
            ones8, ohe, (((1,), (1,)), ((), ())),
            preferred_element_type=jnp.float32))

    acc_num[...] += sum(num_parts)
    acc_den[...] += sum(den_parts)

    @pl.when(i == NBLK - 1)
    def _finish():
        denT = acc_den[...]
        den8 = denT.T
        ones = jnp.full((8, NODE_DIM), 0.125, dtype=jnp.float32)
        den = jax.lax.dot_general(den8, ones,
                                  (((1,), (0,)), ((), ())),
                                  preferred_element_type=jnp.float32)
        den = jnp.where(den == 0.0, 1.0, den)
        out_ref[...] = acc_num[...] / den


@jax.jit
def _pooled(h, batch_i32, W1, b1, W2):
    b3 = batch_i32.reshape(NBLK, 1, BLOCK)
    w2r = W2.reshape(1, HIDDEN_DIM)
    cst = jnp.concatenate([
        jnp.broadcast_to(
            jnp.arange(NUM_GRAPHS, dtype=jnp.float32).reshape(
                NUM_GRAPHS, 1), (NUM_GRAPHS, CHUNK)),
        jnp.broadcast_to(b1.reshape(HIDDEN_DIM, 1),
                         (HIDDEN_DIM, CHUNK)),
    ], axis=0).astype(jnp.bfloat16)
    in_specs = [
        pl.BlockSpec((BLOCK, NODE_DIM), lambda i: (i, 0)),
        pl.BlockSpec((1, 1, BLOCK), lambda i: (i, 0, 0)),
        pl.BlockSpec((HIDDEN_DIM, NODE_DIM), lambda i: (0, 0)),
        pl.BlockSpec((1, HIDDEN_DIM), lambda i: (0, 0)),
        pl.BlockSpec((NUM_GRAPHS + HIDDEN_DIM, CHUNK), lambda i: (0, 0)),
    ]
    return pl.pallas_call(
        _pool_kernel,
        grid=(NBLK,),
        in_specs=in_specs,
        out_specs=pl.BlockSpec((NUM_GRAPHS, NODE_DIM), lambda i: (0, 0)),
        out_shape=jax.ShapeDtypeStruct((NUM_GRAPHS, NODE_DIM), jnp.float32),
        scratch_shapes=[
            pltpu.VMEM((NUM_GRAPHS, NODE_DIM), jnp.float32),
            pltpu.VMEM((8, NUM_GRAPHS), jnp.float32),
        ],
        compiler_params=pltpu.CompilerParams(
            dimension_semantics=("arbitrary",),
        ),
    )(h, b3, W1, w2r, cst)


def kernel(h, batch, W1, b1, W2, b2):
    del b2
    return _pooled(h, batch.astype(jnp.int32), W1, b1, W2)
